# Initial kernel scaffold; baseline (speedup 1.0000x reference)
import jax
import jax.numpy as jnp
from jax.experimental import pallas as pl


def kernel(x, edge_index, edge_attr, W0, a_src0, a_dst0, We0, a_e0, b0, W1, a_src1, a_dst1, We1, a_e1, b1, Ws, a_srcs, a_dsts, bs):
    raise NotImplementedError("write your pallas kernel here")


# speedup vs baseline: 12.1868x; 12.1868x over previous
import jax
import jax.numpy as jnp
from jax import lax
from jax.experimental import pallas as pl
from jax.experimental.pallas import tpu as pltpu
from jax.experimental.pallas import tpu_sc as plsc

NN = 10000
EE = 320000
DD = 128
DQ = DD // 4
NB = 4
NP = 10240
NC = 2
NS = 16
ET = EE // NS
CW = 80
CH = ET // CW
RPT = NP // NS
RB = 512
GRID = NP // RB

_SQRT_HALF = 0.7071067811865476


def _gelu(x):
    return 0.5 * x * (1.0 + lax.erf(x * _SQRT_HALF))


def _mm0_body(x_ref, w_ref, asrc_ref, adst_ref, h_ref, s_ref, d_ref):
    h = jnp.dot(x_ref[...], w_ref[...], preferred_element_type=jnp.float32)
    h_ref[...] = h
    s_ref[...] = jnp.sum(h * asrc_ref[...], axis=1)
    d_ref[...] = jnp.sum(h * adst_ref[...], axis=1)


def _mm0(x, W, asrc, adst):
    return pl.pallas_call(
        _mm0_body,
        grid=(GRID,),
        in_specs=[
            pl.BlockSpec((RB, DD), lambda i: (i, 0)),
            pl.BlockSpec((DD, DD), lambda i: (0, 0)),
            pl.BlockSpec((1, DD), lambda i: (0, 0)),
            pl.BlockSpec((1, DD), lambda i: (0, 0)),
        ],
        out_specs=[
            pl.BlockSpec((RB, DD), lambda i: (i, 0)),
            pl.BlockSpec((RB,), lambda i: (i,)),
            pl.BlockSpec((RB,), lambda i: (i,)),
        ],
        out_shape=[
            jax.ShapeDtypeStruct((NP, DD), jnp.float32),
            jax.ShapeDtypeStruct((NP,), jnp.float32),
            jax.ShapeDtypeStruct((NP,), jnp.float32),
        ],
    )(x, W, asrc, adst)


def _cmm1_body(p_ref, dn_ref, b_ref, w_ref, asrc_ref, adst_ref,
               h_ref, s_ref, d_ref):
    dn = dn_ref[...] + 1e-16
    pc = jnp.concatenate([p_ref[0], p_ref[1], p_ref[2], p_ref[3]], axis=1)
    g = _gelu(pc / dn + b_ref[...])
    h = jnp.dot(g, w_ref[...], preferred_element_type=jnp.float32)
    h_ref[...] = h
    s_ref[...] = jnp.sum(h * asrc_ref[...], axis=1)
    d_ref[...] = jnp.sum(h * adst_ref[...], axis=1)


def _cmm1(p, dn, b, W, asrc, adst):
    return pl.pallas_call(
        _cmm1_body,
        grid=(GRID,),
        in_specs=[
            pl.BlockSpec((NB, RB, DQ), lambda i: (0, i, 0)),
            pl.BlockSpec((RB, 1), lambda i: (i, 0)),
            pl.BlockSpec((1, DD), lambda i: (0, 0)),
            pl.BlockSpec((DD, DD), lambda i: (0, 0)),
            pl.BlockSpec((1, DD), lambda i: (0, 0)),
            pl.BlockSpec((1, DD), lambda i: (0, 0)),
        ],
        out_specs=[
            pl.BlockSpec((RB, DD), lambda i: (i, 0)),
            pl.BlockSpec((RB,), lambda i: (i,)),
            pl.BlockSpec((RB,), lambda i: (i,)),
        ],
        out_shape=[
            jax.ShapeDtypeStruct((NP, DD), jnp.float32),
            jax.ShapeDtypeStruct((NP,), jnp.float32),
            jax.ShapeDtypeStruct((NP,), jnp.float32),
        ],
    )(p, dn, b, W, asrc, adst)


def _cmm2_body(p_ref, dn_ref, b_ref, x0_ref, wa_ref, wb_ref,
               asrc_ref, adst_ref, h_ref, s_ref, d_ref):
    dn = dn_ref[...] + 1e-16
    pc = jnp.concatenate([p_ref[0], p_ref[1], p_ref[2], p_ref[3]], axis=1)
    g = _gelu(pc / dn + b_ref[...])
    h = (jnp.dot(x0_ref[...], wa_ref[...], preferred_element_type=jnp.float32)
         + jnp.dot(g, wb_ref[...], preferred_element_type=jnp.float32))
    h_ref[...] = h
    s_ref[...] = jnp.sum(h * asrc_ref[...], axis=1)
    d_ref[...] = jnp.sum(h * adst_ref[...], axis=1)


def _cmm2(p, dn, b, x0, wa, wb, asrc, adst):
    return pl.pallas_call(
        _cmm2_body,
        grid=(GRID,),
        in_specs=[
            pl.BlockSpec((NB, RB, DQ), lambda i: (0, i, 0)),
            pl.BlockSpec((RB, 1), lambda i: (i, 0)),
            pl.BlockSpec((1, DD), lambda i: (0, 0)),
            pl.BlockSpec((RB, DD), lambda i: (i, 0)),
            pl.BlockSpec((DD, DD), lambda i: (0, 0)),
            pl.BlockSpec((DD, DD), lambda i: (0, 0)),
            pl.BlockSpec((1, DD), lambda i: (0, 0)),
            pl.BlockSpec((1, DD), lambda i: (0, 0)),
        ],
        out_specs=[
            pl.BlockSpec((RB, DD), lambda i: (i, 0)),
            pl.BlockSpec((RB,), lambda i: (i,)),
            pl.BlockSpec((RB,), lambda i: (i,)),
        ],
        out_shape=[
            jax.ShapeDtypeStruct((NP, DD), jnp.float32),
            jax.ShapeDtypeStruct((NP,), jnp.float32),
            jax.ShapeDtypeStruct((NP,), jnp.float32),
        ],
    )(p, dn, b, x0, wa, wb, asrc, adst)


def _final_body(p_ref, dn_ref, b_ref, o_ref):
    dn = dn_ref[...] + 1e-16
    pc = jnp.concatenate([p_ref[0], p_ref[1], p_ref[2], p_ref[3]], axis=1)
    o_ref[...] = pc / dn + b_ref[...]


def _final(p, dn, b):
    return pl.pallas_call(
        _final_body,
        grid=(GRID,),
        in_specs=[
            pl.BlockSpec((NB, RB, DQ), lambda i: (0, i, 0)),
            pl.BlockSpec((RB, 1), lambda i: (i, 0)),
            pl.BlockSpec((1, DD), lambda i: (0, 0)),
        ],
        out_specs=pl.BlockSpec((RB, DD), lambda i: (i, 0)),
        out_shape=jax.ShapeDtypeStruct((NP, DD), jnp.float32),
    )(p, dn, b)


def _ealpha_body(ea0_ref, ea1_ref, we0_ref, ae0_ref, we1_ref, ae1_ref,
                 o0_ref, o1_ref):
    c0 = jnp.sum(we0_ref[...] * ae0_ref[...], axis=1, keepdims=True)
    c1 = jnp.sum(we1_ref[...] * ae1_ref[...], axis=1, keepdims=True)
    o0_ref[...] = ea0_ref[...] * c0[0:1] + ea1_ref[...] * c0[1:2]
    o1_ref[...] = ea0_ref[...] * c1[0:1] + ea1_ref[...] * c1[1:2]


def _ealpha(ea0, ea1, we0, ae0, we1, ae1):
    eb = EE // DD
    return pl.pallas_call(
        _ealpha_body,
        out_shape=[
            jax.ShapeDtypeStruct((eb, DD), jnp.float32),
            jax.ShapeDtypeStruct((eb, DD), jnp.float32),
        ],
    )(ea0, ea1, we0, ae0, we1, ae1)


def _sc_edge_factory():
    mesh = plsc.VectorSubcoreMesh(core_axis_name="c", subcore_axis_name="s")
    scratch = [
        pltpu.VMEM((NP,), jnp.float32),
        pltpu.VMEM((NP,), jnp.float32),
        pltpu.VMEM((CH, CW), jnp.int32),
        pltpu.VMEM((CH, CW), jnp.int32),
        pltpu.VMEM((CH, CW), jnp.float32),
    ]
    scratch += [
        pltpu.VMEM((CH, CW), jnp.float32),
        pltpu.VMEM((CW, DQ), jnp.float32),
        pltpu.VMEM((40, DQ), jnp.float32),
        pltpu.VMEM((RPT,), jnp.float32),
        pltpu.VMEM_SHARED((NP, DQ), jnp.float32),
        pltpu.VMEM_SHARED((NP,), jnp.float32),
        pltpu.SemaphoreType.DMA,
    ]
    out_type = (
        jax.ShapeDtypeStruct((NB, NP, DQ), jnp.float32),
        jax.ShapeDtypeStruct((NP,), jnp.float32),
    )

    def body(*refs):
        (h4_hbm, s_hbm, d_hbm, srcm_hbm, dstm_hbm, eal_hbm,
         parts_hbm, den_hbm,
         s_v, d_v, src_m, dst_m, ea_m, ex_m, rows_v, zrows_v, zden_v,
         acc_sh, den_sh, sem) = refs
        c = lax.axis_index("c")
        sid = lax.axis_index("s")

        pltpu.sync_copy(s_hbm, s_v)
        pltpu.sync_copy(d_hbm, d_v)
        pltpu.sync_copy(srcm_hbm.at[sid], src_m)
        pltpu.sync_copy(dstm_hbm.at[sid], dst_m)
        pltpu.sync_copy(eal_hbm.at[sid], ea_m)

        z16 = jnp.zeros((16,), jnp.float32)

        def zrow(i, carry):
            for q in range(DQ // 16):
                zrows_v[i, pl.ds(q * 16, 16)] = z16
            return carry
        lax.fori_loop(0, 40, zrow, 0)

        def zden(i, carry):
            zden_v[pl.ds(i * 16, 16)] = z16
            return carry
        lax.fori_loop(0, RPT // 16, zden, 0)

        base = sid * RPT

        for p in range(2):
            blk = 2 * p + c
            hc_hbm = h4_hbm.at[blk]
            for t in range(RPT // 40):
                pltpu.sync_copy(zrows_v, acc_sh.at[pl.ds(base + t * 40, 40)])
            if p == 0:
                @pl.when(c == 0)
                def _():
                    pltpu.sync_copy(zden_v, den_sh.at[pl.ds(base, RPT)])
            plsc.subcore_barrier()

            def chunk(j, carry):
                pltpu.async_copy(hc_hbm.at[src_m.at[j]], rows_v, sem).wait()
                if p == 0:
                    for k in range(CW // 16):
                        sl = pl.ds(k * 16, 16)
                        a = (plsc.load_gather(s_v, [src_m[j, sl]])
                             + plsc.load_gather(d_v, [dst_m[j, sl]])
                             + ea_m[j, sl])
                        a = jnp.where(a >= 0, a, 0.2 * a)
                        ex_m[j, sl] = jnp.exp(a)
                bj = jnp.broadcast_to(j, (16,)).astype(jnp.int32)

                def scale(e, carry2):
                    be = jnp.broadcast_to(e, (16,)).astype(jnp.int32)
                    spl = plsc.load_gather(ex_m, [bj, be])
                    for q in range(DQ // 16):
                        sl2 = pl.ds(q * 16, 16)
                        rows_v[e, sl2] = rows_v[e, sl2] * spl
                    return carry2
                lax.fori_loop(0, CW, scale, 0, unroll=4)

                pltpu.sync_copy(rows_v, acc_sh.at[dst_m.at[j]], add=True)

                if p == 0:
                    @pl.when(c == 0)
                    def _():
                        pltpu.sync_copy(ex_m.at[j], den_sh.at[dst_m.at[j]],
                                        add=True)
                return carry
            lax.fori_loop(0, CH, chunk, 0)
            plsc.subcore_barrier()

            pltpu.sync_copy(acc_sh.at[pl.ds(base, RPT)],
                            parts_hbm.at[blk, pl.ds(base, RPT)])

        @pl.when(c == 0)
        def _():
            pltpu.sync_copy(den_sh.at[pl.ds(base, RPT)],
                            den_hbm.at[pl.ds(base, RPT)])

    return pl.kernel(body, out_type=out_type, mesh=mesh,
                     scratch_types=scratch,
                     compiler_params=pltpu.CompilerParams(
                         needs_layout_passes=False,
                         use_tc_tiling_on_sc=False))


_sc_edge = _sc_edge_factory()


@jax.jit
def _forward(x, edge_index, edge_attr, W0, a_src0, a_dst0, We0, a_e0, b0,
             W1, a_src1, a_dst1, We1, a_e1, b1, Ws, a_srcs, a_dsts, bs):
    xp = jnp.zeros((NP, DD), jnp.float32).at[:NN].set(x)
    srcm = edge_index[0].reshape(NS, CH, CW)
    dstm = edge_index[1].reshape(NS, CH, CW)
    ea0 = edge_attr[:, 0].reshape(EE // DD, DD)
    ea1 = edge_attr[:, 1].reshape(EE // DD, DD)
    we0 = jnp.zeros((8, DD), jnp.float32).at[:2].set(We0)
    we1 = jnp.zeros((8, DD), jnp.float32).at[:2].set(We1)
    eal0, eal1 = _ealpha(ea0, ea1, we0, a_e0.reshape(1, DD),
                         we1, a_e1.reshape(1, DD))
    eal0 = eal0.reshape(NS, CH, CW)
    eal1 = eal1.reshape(NS, CH, CW)

    eal_all = jnp.stack([eal0, eal1, jnp.zeros_like(eal0)], axis=0)

    def split(h):
        return jnp.stack([h[:, q * DQ:(q + 1) * DQ] for q in range(NB)],
                         axis=0)

    h0, s0, d0 = _mm0(xp, W0, a_src0.reshape(1, DD), a_dst0.reshape(1, DD))

    def step(i, carry):
        h2, s, d, out = carry
        eal = lax.dynamic_index_in_dim(eal_all, i, 0, keepdims=False)
        p, dn = _sc_edge(h2, s, d, srcm, dstm, eal)
        dnr = dn.reshape(NP, 1)

        def br0(_):
            h, s2, d2 = _cmm1(p, dnr, b0.reshape(1, DD), W1,
                              a_src1.reshape(1, DD), a_dst1.reshape(1, DD))
            return (split(h), s2, d2, out)

        def br1(_):
            h, s2, d2 = _cmm2(p, dnr, b1.reshape(1, DD), xp,
                              Ws[:DD], Ws[DD:],
                              a_srcs.reshape(1, DD), a_dsts.reshape(1, DD))
            return (split(h), s2, d2, out)

        def br2(_):
            o = _final(p, dnr, bs.reshape(1, DD))
            return (h2, s, d, o)

        return lax.switch(i, [br0, br1, br2], None)

    carry = (split(h0), s0, d0, jnp.zeros((NP, DD), jnp.float32))
    _, _, _, out = lax.fori_loop(0, 3, step, carry)
    return out[:NN]


def kernel(x, edge_index, edge_attr, W0, a_src0, a_dst0, We0, a_e0, b0,
           W1, a_src1, a_dst1, We1, a_e1, b1, Ws, a_srcs, a_dsts, bs):
    return _forward(x, edge_index, edge_attr, W0, a_src0, a_dst0, We0, a_e0,
                    b0, W1, a_src1, a_dst1, We1, a_e1, b1, Ws, a_srcs,
                    a_dsts, bs)


# speedup vs baseline: 24.2897x; 1.9931x over previous
import jax
import jax.numpy as jnp
from jax import lax
from jax.experimental import pallas as pl
from jax.experimental.pallas import tpu as pltpu
from jax.experimental.pallas import tpu_sc as plsc

NN = 10000
EE = 320000
DD = 128
DQ = DD // 4
NB = 4
NP = 10240
NC = 2
NS = 16
ET = EE // NS
CW = 80
CH = ET // CW
SUP = 5
NSC = CH // SUP
RPT = NP // NS
RB = 512
GRID = NP // RB

_SQRT_HALF = 0.7071067811865476


def _gelu(x):
    return 0.5 * x * (1.0 + lax.erf(x * _SQRT_HALF))


def _mm0_body(x_ref, w_ref, asrc_ref, adst_ref, h_ref, s_ref, d_ref):
    h = jnp.dot(x_ref[...], w_ref[...], preferred_element_type=jnp.float32)
    for q in range(NB):
        h_ref[q] = h[:, q * DQ:(q + 1) * DQ]
    s_ref[...] = jnp.sum(h * asrc_ref[...], axis=1)
    d_ref[...] = jnp.sum(h * adst_ref[...], axis=1)


def _mm0(x, W, asrc, adst):
    return pl.pallas_call(
        _mm0_body,
        grid=(GRID,),
        in_specs=[
            pl.BlockSpec((RB, DD), lambda i: (i, 0)),
            pl.BlockSpec((DD, DD), lambda i: (0, 0)),
            pl.BlockSpec((1, DD), lambda i: (0, 0)),
            pl.BlockSpec((1, DD), lambda i: (0, 0)),
        ],
        out_specs=[
            pl.BlockSpec((NB, RB, DQ), lambda i: (0, i, 0)),
            pl.BlockSpec((RB,), lambda i: (i,)),
            pl.BlockSpec((RB,), lambda i: (i,)),
        ],
        out_shape=[
            jax.ShapeDtypeStruct((NB, NP, DQ), jnp.float32),
            jax.ShapeDtypeStruct((NP,), jnp.float32),
            jax.ShapeDtypeStruct((NP,), jnp.float32),
        ],
    )(x, W, asrc, adst)


def _cmm1_body(p_ref, dn_ref, b_ref, w_ref, asrc_ref, adst_ref,
               h_ref, s_ref, d_ref):
    dn = dn_ref[...] + 1e-16
    pc = jnp.concatenate([p_ref[0], p_ref[1], p_ref[2], p_ref[3]], axis=1)
    g = _gelu(pc / dn + b_ref[...])
    h = jnp.dot(g, w_ref[...], preferred_element_type=jnp.float32)
    for q in range(NB):
        h_ref[q] = h[:, q * DQ:(q + 1) * DQ]
    s_ref[...] = jnp.sum(h * asrc_ref[...], axis=1)
    d_ref[...] = jnp.sum(h * adst_ref[...], axis=1)


def _cmm1(p, dn, b, W, asrc, adst):
    return pl.pallas_call(
        _cmm1_body,
        grid=(GRID,),
        in_specs=[
            pl.BlockSpec((NB, RB, DQ), lambda i: (0, i, 0)),
            pl.BlockSpec((RB, 1), lambda i: (i, 0)),
            pl.BlockSpec((1, DD), lambda i: (0, 0)),
            pl.BlockSpec((DD, DD), lambda i: (0, 0)),
            pl.BlockSpec((1, DD), lambda i: (0, 0)),
            pl.BlockSpec((1, DD), lambda i: (0, 0)),
        ],
        out_specs=[
            pl.BlockSpec((NB, RB, DQ), lambda i: (0, i, 0)),
            pl.BlockSpec((RB,), lambda i: (i,)),
            pl.BlockSpec((RB,), lambda i: (i,)),
        ],
        out_shape=[
            jax.ShapeDtypeStruct((NB, NP, DQ), jnp.float32),
            jax.ShapeDtypeStruct((NP,), jnp.float32),
            jax.ShapeDtypeStruct((NP,), jnp.float32),
        ],
    )(p, dn, b, W, asrc, adst)


def _cmm2_body(p_ref, dn_ref, b_ref, x0_ref, wa_ref, wb_ref,
               asrc_ref, adst_ref, h_ref, s_ref, d_ref):
    dn = dn_ref[...] + 1e-16
    pc = jnp.concatenate([p_ref[0], p_ref[1], p_ref[2], p_ref[3]], axis=1)
    g = _gelu(pc / dn + b_ref[...])
    h = (jnp.dot(x0_ref[...], wa_ref[...], preferred_element_type=jnp.float32)
         + jnp.dot(g, wb_ref[...], preferred_element_type=jnp.float32))
    for q in range(NB):
        h_ref[q] = h[:, q * DQ:(q + 1) * DQ]
    s_ref[...] = jnp.sum(h * asrc_ref[...], axis=1)
    d_ref[...] = jnp.sum(h * adst_ref[...], axis=1)


def _cmm2(p, dn, b, x0, wa, wb, asrc, adst):
    return pl.pallas_call(
        _cmm2_body,
        grid=(GRID,),
        in_specs=[
            pl.BlockSpec((NB, RB, DQ), lambda i: (0, i, 0)),
            pl.BlockSpec((RB, 1), lambda i: (i, 0)),
            pl.BlockSpec((1, DD), lambda i: (0, 0)),
            pl.BlockSpec((RB, DD), lambda i: (i, 0)),
            pl.BlockSpec((DD, DD), lambda i: (0, 0)),
            pl.BlockSpec((DD, DD), lambda i: (0, 0)),
            pl.BlockSpec((1, DD), lambda i: (0, 0)),
            pl.BlockSpec((1, DD), lambda i: (0, 0)),
        ],
        out_specs=[
            pl.BlockSpec((NB, RB, DQ), lambda i: (0, i, 0)),
            pl.BlockSpec((RB,), lambda i: (i,)),
            pl.BlockSpec((RB,), lambda i: (i,)),
        ],
        out_shape=[
            jax.ShapeDtypeStruct((NB, NP, DQ), jnp.float32),
            jax.ShapeDtypeStruct((NP,), jnp.float32),
            jax.ShapeDtypeStruct((NP,), jnp.float32),
        ],
    )(p, dn, b, x0, wa, wb, asrc, adst)


def _final_body(p_ref, dn_ref, b_ref, o_ref):
    dn = dn_ref[...] + 1e-16
    pc = jnp.concatenate([p_ref[0], p_ref[1], p_ref[2], p_ref[3]], axis=1)
    o_ref[...] = pc / dn + b_ref[...]


def _final(p, dn, b):
    return pl.pallas_call(
        _final_body,
        grid=(GRID,),
        in_specs=[
            pl.BlockSpec((NB, RB, DQ), lambda i: (0, i, 0)),
            pl.BlockSpec((RB, 1), lambda i: (i, 0)),
            pl.BlockSpec((1, DD), lambda i: (0, 0)),
        ],
        out_specs=pl.BlockSpec((RB, DD), lambda i: (i, 0)),
        out_shape=jax.ShapeDtypeStruct((NP, DD), jnp.float32),
    )(p, dn, b)


def _ealpha_body(ea0_ref, ea1_ref, we0_ref, ae0_ref, we1_ref, ae1_ref,
                 o0_ref, o1_ref):
    c0 = jnp.sum(we0_ref[...] * ae0_ref[...], axis=1, keepdims=True)
    c1 = jnp.sum(we1_ref[...] * ae1_ref[...], axis=1, keepdims=True)
    o0_ref[...] = ea0_ref[...] * c0[0:1] + ea1_ref[...] * c0[1:2]
    o1_ref[...] = ea0_ref[...] * c1[0:1] + ea1_ref[...] * c1[1:2]


def _ealpha(ea0, ea1, we0, ae0, we1, ae1):
    eb = EE // DD
    return pl.pallas_call(
        _ealpha_body,
        out_shape=[
            jax.ShapeDtypeStruct((eb, DD), jnp.float32),
            jax.ShapeDtypeStruct((eb, DD), jnp.float32),
        ],
    )(ea0, ea1, we0, ae0, we1, ae1)


def _sc_edge_factory():
    mesh = plsc.VectorSubcoreMesh(core_axis_name="c", subcore_axis_name="s")
    scratch = [
        pltpu.VMEM((NP,), jnp.float32),
        pltpu.VMEM((NP,), jnp.float32),
        pltpu.VMEM((CH, CW), jnp.int32),
        pltpu.VMEM((CH, CW), jnp.int32),
    ]
    scratch += [
        pltpu.VMEM((2, SUP, CW), jnp.float32),
        pltpu.VMEM((2, SUP, CW), jnp.float32),
        pltpu.VMEM((CH, CW), jnp.float32),
        pltpu.VMEM((2, SUP * CW, DQ), jnp.float32),
        pltpu.VMEM((40, DQ), jnp.float32),
        pltpu.VMEM((RPT,), jnp.float32),
        pltpu.VMEM_SHARED((NP, DQ), jnp.float32),
        pltpu.VMEM_SHARED((NP,), jnp.float32),
        pltpu.SemaphoreType.DMA,
        pltpu.SemaphoreType.DMA,
    ]
    out_type = (
        jax.ShapeDtypeStruct((NB, NP, DQ), jnp.float32),
        jax.ShapeDtypeStruct((NP,), jnp.float32),
    )

    def body(*refs):
        (h4_hbm, s_hbm, d_hbm, srcm_hbm, dstm_hbm, eal_hbm,
         parts_hbm, den_hbm,
         s_v, d_v, src_m, dst_m, ea_c, ex_c, ex_full, rows_v, zrows_v,
         zden_v, acc_sh, den_sh, sem, sem2) = refs
        c = lax.axis_index("c")
        sid = lax.axis_index("s")

        stage = [
            pltpu.async_copy(s_hbm, s_v, sem),
            pltpu.async_copy(d_hbm, d_v, sem),
            pltpu.async_copy(srcm_hbm.at[sid], src_m, sem),
            pltpu.async_copy(dstm_hbm.at[sid], dst_m, sem),
        ]
        for cp in stage:
            cp.wait()

        z16 = jnp.zeros((16,), jnp.float32)

        def zrow(i, carry):
            for q in range(DQ // 16):
                zrows_v[i, pl.ds(q * 16, 16)] = z16
            return carry
        lax.fori_loop(0, 40, zrow, 0)

        def zden(i, carry):
            zden_v[pl.ds(i * 16, 16)] = z16
            return carry
        lax.fori_loop(0, RPT // 16, zden, 0)

        base = sid * RPT

        for p in range(2):
            blk = 2 * p + c
            hc_hbm = h4_hbm.at[blk]
            zcps = [
                pltpu.async_copy(zrows_v, acc_sh.at[pl.ds(base + t * 40, 40)],
                                 sem)
                for t in range(RPT // 40)
            ]
            for cp in zcps:
                cp.wait()
            if p == 0:
                @pl.when(c == 0)
                def _():
                    pltpu.sync_copy(zden_v, den_sh.at[pl.ds(base, RPT)])
            plsc.subcore_barrier()

            for k in range(SUP):
                pltpu.async_copy(hc_hbm.at[src_m.at[k]],
                                 rows_v.at[0, pl.ds(k * CW, CW)], sem)
            if p == 0:
                pltpu.async_copy(eal_hbm.at[sid, pl.ds(0, SUP)],
                                 ea_c.at[0], sem)

            def wait_scatters(Sprev, bprev):
                for k in range(SUP):
                    jp = Sprev * SUP + k
                    pltpu.make_async_copy(
                        rows_v.at[bprev, pl.ds(k * CW, CW)],
                        acc_sh.at[dst_m.at[jp]], sem2).wait()
                if p == 0:
                    @pl.when(c == 0)
                    def _():
                        for k in range(SUP):
                            jp = Sprev * SUP + k
                            pltpu.make_async_copy(
                                ex_c.at[bprev, k],
                                den_sh.at[dst_m.at[jp]], sem2).wait()

            def sup_body(S, carry):
                b = lax.rem(S, 2)
                for k in range(SUP):
                    j = S * SUP + k
                    pltpu.make_async_copy(
                        hc_hbm.at[src_m.at[j]],
                        rows_v.at[b, pl.ds(k * CW, CW)], sem).wait()
                if p == 0:
                    pltpu.make_async_copy(
                        eal_hbm.at[sid, pl.ds(S * SUP, SUP)],
                        ea_c.at[b], sem).wait()

                @pl.when(S >= 1)
                def _():
                    wait_scatters(S - 1, 1 - b)

                @pl.when(S + 1 < NSC)
                def _():
                    for k in range(SUP):
                        jn = (S + 1) * SUP + k
                        pltpu.async_copy(
                            hc_hbm.at[src_m.at[jn]],
                            rows_v.at[1 - b, pl.ds(k * CW, CW)], sem)
                    if p == 0:
                        pltpu.async_copy(
                            eal_hbm.at[sid, pl.ds((S + 1) * SUP, SUP)],
                            ea_c.at[1 - b], sem)

                for k in range(SUP):
                    j = S * SUP + k
                    if p == 0:
                        for t16 in range(CW // 16):
                            sl = pl.ds(t16 * 16, 16)
                            a = (plsc.load_gather(s_v, [src_m[j, sl]])
                                 + plsc.load_gather(d_v, [dst_m[j, sl]])
                                 + ea_c[b, k, sl])
                            a = jnp.where(a >= 0, a, 0.2 * a)
                            ex = jnp.exp(a)
                            ex_c[b, k, sl] = ex
                            ex_full[j, sl] = ex
                    koff = k * CW
                    bj = jnp.broadcast_to(j, (16,)).astype(jnp.int32)

                    def scale(e, carry2):
                        be = jnp.broadcast_to(e, (16,)).astype(jnp.int32)
                        spl = plsc.load_gather(ex_full, [bj, be])
                        for q in range(DQ // 16):
                            sl2 = pl.ds(q * 16, 16)
                            rows_v[b, koff + e, sl2] = (
                                rows_v[b, koff + e, sl2] * spl)
                        return carry2
                    lax.fori_loop(0, CW, scale, 0, unroll=4)

                for k in range(SUP):
                    j = S * SUP + k
                    pltpu.async_copy(rows_v.at[b, pl.ds(k * CW, CW)],
                                     acc_sh.at[dst_m.at[j]], sem2, add=True)
                    if p == 0:
                        @pl.when(c == 0)
                        def _():
                            pltpu.async_copy(ex_c.at[b, k],
                                             den_sh.at[dst_m.at[j]],
                                             sem2, add=True)
                return carry
            lax.fori_loop(0, NSC, sup_body, 0)
            wait_scatters(NSC - 1, (NSC - 1) % 2)
            plsc.subcore_barrier()

            pltpu.sync_copy(acc_sh.at[pl.ds(base, RPT)],
                            parts_hbm.at[blk, pl.ds(base, RPT)])

        @pl.when(c == 0)
        def _():
            pltpu.sync_copy(den_sh.at[pl.ds(base, RPT)],
                            den_hbm.at[pl.ds(base, RPT)])

    return pl.kernel(body, out_type=out_type, mesh=mesh,
                     scratch_types=scratch,
                     compiler_params=pltpu.CompilerParams(
                         needs_layout_passes=False,
                         use_tc_tiling_on_sc=False))


_sc_edge = _sc_edge_factory()


@jax.jit
def _forward(x, edge_index, edge_attr, W0, a_src0, a_dst0, We0, a_e0, b0,
             W1, a_src1, a_dst1, We1, a_e1, b1, Ws, a_srcs, a_dsts, bs):
    xp = jnp.zeros((NP, DD), jnp.float32).at[:NN].set(x)
    srcm = edge_index[0].reshape(NS, CH, CW)
    dstm = edge_index[1].reshape(NS, CH, CW)
    ea0 = edge_attr[:, 0].reshape(EE // DD, DD)
    ea1 = edge_attr[:, 1].reshape(EE // DD, DD)
    we0 = jnp.zeros((8, DD), jnp.float32).at[:2].set(We0)
    we1 = jnp.zeros((8, DD), jnp.float32).at[:2].set(We1)
    eal0, eal1 = _ealpha(ea0, ea1, we0, a_e0.reshape(1, DD),
                         we1, a_e1.reshape(1, DD))
    eal0 = eal0.reshape(NS, CH, CW)
    eal1 = eal1.reshape(NS, CH, CW)

    eal_all = jnp.stack([eal0, eal1, jnp.zeros_like(eal0)], axis=0)

    h0, s0, d0 = _mm0(xp, W0, a_src0.reshape(1, DD), a_dst0.reshape(1, DD))

    def step(i, carry):
        h2, s, d, out = carry
        eal = lax.dynamic_index_in_dim(eal_all, i, 0, keepdims=False)
        p, dn = _sc_edge(h2, s, d, srcm, dstm, eal)
        dnr = dn.reshape(NP, 1)

        def br0(_):
            h, s2, d2 = _cmm1(p, dnr, b0.reshape(1, DD), W1,
                              a_src1.reshape(1, DD), a_dst1.reshape(1, DD))
            return (h, s2, d2, out)

        def br1(_):
            h, s2, d2 = _cmm2(p, dnr, b1.reshape(1, DD), xp,
                              Ws[:DD], Ws[DD:],
                              a_srcs.reshape(1, DD), a_dsts.reshape(1, DD))
            return (h, s2, d2, out)

        def br2(_):
            o = _final(p, dnr, bs.reshape(1, DD))
            return (h2, s, d, o)

        return lax.switch(i, [br0, br1, br2], None)

    carry = (h0, s0, d0, jnp.zeros((NP, DD), jnp.float32))
    _, _, _, out = lax.fori_loop(0, 3, step, carry)
    return out[:NN]


def kernel(x, edge_index, edge_attr, W0, a_src0, a_dst0, We0, a_e0, b0,
           W1, a_src1, a_dst1, We1, a_e1, b1, Ws, a_srcs, a_dsts, bs):
    return _forward(x, edge_index, edge_attr, W0, a_src0, a_dst0, We0, a_e0,
                    b0, W1, a_src1, a_dst1, We1, a_e1, b1, Ws, a_srcs,
                    a_dsts, bs)
